# Initial kernel scaffold; baseline (speedup 1.0000x reference)
import jax
import jax.numpy as jnp
from jax.experimental import pallas as pl


def kernel(shp, intrinsics, frustum_masks):
    raise NotImplementedError("write your pallas kernel here")


# speedup vs baseline: 3.6492x; 3.6492x over previous
import jax
import jax.numpy as jnp
from jax.experimental import pallas as pl

_IMAGE_SIZE = (160, 120)
_DEPTH_MIN = 0.4
_DEPTH_MAX = 6.0
_VOXEL_SIZE = 0.03
_GRID_DIMENSIONS = jnp.array([256.0, 256.0, 256.0], dtype=jnp.float32)

_N = 256
_M = _N * 5
_IB = 8


def _gen_frustum(image_size, intrinsic_inv, depth_min, depth_max):
    x = float(image_size[0])
    y = float(image_size[1])
    eight_points = jnp.array([
        [0.0 * depth_min, 0.0 * depth_min, depth_min, 1.0],
        [0.0 * depth_min, y * depth_min, depth_min, 1.0],
        [x * depth_min, y * depth_min, depth_min, 1.0],
        [x * depth_min, 0.0 * depth_min, depth_min, 1.0],
        [0.0 * depth_max, 0.0 * depth_max, depth_max, 1.0],
        [0.0 * depth_max, y * depth_max, depth_max, 1.0],
        [x * depth_max, y * depth_max, depth_max, 1.0],
        [x * depth_max, 0.0 * depth_max, depth_max, 1.0]], dtype=jnp.float32).T
    frustum = (intrinsic_inv @ eight_points).T
    return frustum[:, :3]


def _gen_frustum_volume(frustum, voxel_size):
    maxs = jnp.max(frustum, axis=0) / voxel_size
    mins = jnp.min(frustum, axis=0) / voxel_size
    dims = jnp.ceil(maxs - mins)
    camera2frustum = jnp.array(
        [[1.0 / voxel_size, 0.0, 0.0, 0.0],
         [0.0, 1.0 / voxel_size, 0.0, 0.0],
         [0.0, 0.0, 1.0 / voxel_size, 0.0],
         [0.0, 0.0, 0.0, 1.0]], dtype=jnp.float32)
    camera2frustum = camera2frustum.at[:3, 3].set(-mins)
    return dims, camera2frustum


def _tables(bi, shp, intrinsic):
    intrinsic_inv = jnp.linalg.inv(intrinsic)
    frustum = _gen_frustum(_IMAGE_SIZE, intrinsic_inv, _DEPTH_MIN, _DEPTH_MAX)
    _, camera2frustum = _gen_frustum_volume(frustum, _VOXEL_SIZE)
    dims, _ = _gen_frustum_volume(frustum, _VOXEL_SIZE)
    padding = (_GRID_DIMENSIONS - dims) / 2.0
    minv = jnp.linalg.inv(camera2frustum)

    ar = jnp.arange(_N, dtype=jnp.float32)
    g0 = (256.0 - ar) - padding[0] - 1.0
    g1 = (256.0 - ar) - padding[1] - 1.0
    g2 = ar - padding[2] - 1.0
    z = jnp.zeros_like(g0)
    o = jnp.ones_like(g0)
    gh = jnp.concatenate([jnp.stack([g0, z, z, o]),
                          jnp.stack([z, g1, z, o]),
                          jnp.stack([z, z, g2, o])], axis=1)
    pcs = minv @ gh
    pc0, pc1, pc2 = pcs[0, :_N], pcs[1, _N:2 * _N], pcs[2, 2 * _N:]
    cols = jnp.stack([
        jnp.repeat(pc0, _N), jnp.repeat(pc1, _N),
        jnp.tile(pc2, _N), jnp.ones(_N * _N, jnp.float32)])
    dps = intrinsic @ cols
    depth = dps[2].reshape(_N, _N)[0]
    coordx = (dps[0] / dps[2]).reshape(_N, _N)
    coordy = (dps[1] / dps[2]).reshape(_N, _N)

    dok = (depth >= _DEPTH_MIN) & (depth <= _DEPTH_MAX)
    w = shp.astype(jnp.float32)
    cxok = coordx < w[1]
    cyok = coordy < w[0]
    f32 = jnp.float32
    ckx = (dok[None, :] & cxok).astype(f32)
    cky = cyok.astype(f32)
    kx = (dok[None, :] & cxok & (coordx >= 0.0)).astype(f32)
    ky = (cyok & (coordy >= 0.0)).astype(f32)

    m = jnp.arange(_M)
    kk = m // 5
    cc = m % 5
    base_m = jnp.where(cc == 3, kk.astype(f32),
                       jnp.where(cc == 4, depth[kk],
                                 jnp.where(cc == 0, jnp.float32(bi), 0.0)))
    xt = jnp.where((cc == 1)[None, :], coordx[:, kk], base_m[None, :])
    yt = jnp.where((cc == 2)[None, :], coordy[:, kk], 0.0)
    return xt, yt, ckx, cky, kx, ky


def _body(mask_ref, x_ref, y_ref, ckx_ref, cky_ref, kx_ref, ky_ref, e_ref,
          out_ref, kept_ref):
    mask = mask_ref[0]
    ckx = ckx_ref[0]
    cky = cky_ref[0]
    condk = jnp.where(mask, ckx[:, None, :] * cky[None, :, :], 0.0)
    me = jax.lax.dot_general(
        condk.astype(jnp.bfloat16).reshape(_IB * _N, _N), e_ref[...],
        (((1,), (0,)), ((), ())), preferred_element_type=jnp.float32)
    val = x_ref[0][:, None, :] + y_ref[0][None, :, :]
    out_ref[0] = jnp.where(me.reshape(_IB, _N, _M) > 0.5, val, -1.0)
    kept_ref[0] = mask & (kx_ref[0][:, None, :] > 0.5) & (ky_ref[0][None, :, :] > 0.5)


def kernel(shp, intrinsics, frustum_masks):
    b = intrinsics.shape[0]
    tabs = [_tables(bi, shp, intrinsics[bi]) for bi in range(b)]
    xt, yt, ckx, cky, kx, ky = (jnp.stack([t[i] for t in tabs]) for i in range(6))
    e = (jnp.arange(_N)[:, None] == (jnp.arange(_M) // 5)[None, :]).astype(jnp.bfloat16)

    grid = (b, _N // _IB)
    out, kept = pl.pallas_call(
        _body,
        grid=grid,
        in_specs=[
            pl.BlockSpec((1, _IB, _N, _N), lambda bb, ii: (bb, ii, 0, 0)),
            pl.BlockSpec((1, _IB, _M), lambda bb, ii: (bb, ii, 0)),
            pl.BlockSpec((1, _N, _M), lambda bb, ii: (bb, 0, 0)),
            pl.BlockSpec((1, _IB, _N), lambda bb, ii: (bb, ii, 0)),
            pl.BlockSpec((1, _N, _N), lambda bb, ii: (bb, 0, 0)),
            pl.BlockSpec((1, _IB, _N), lambda bb, ii: (bb, ii, 0)),
            pl.BlockSpec((1, _N, _N), lambda bb, ii: (bb, 0, 0)),
            pl.BlockSpec((_N, _M), lambda bb, ii: (0, 0)),
        ],
        out_specs=[
            pl.BlockSpec((1, _IB, _N, _M), lambda bb, ii: (bb, ii, 0, 0)),
            pl.BlockSpec((1, _IB, _N, _N), lambda bb, ii: (bb, ii, 0, 0)),
        ],
        out_shape=[
            jax.ShapeDtypeStruct((b, _N, _N, _M), jnp.float32),
            jax.ShapeDtypeStruct((b, _N, _N, _N), jnp.bool_),
        ],
    )(frustum_masks, xt, yt, ckx, cky, kx, ky, e)
    mappings = out.reshape(b, _N, _N, _N, 5)
    return kept, mappings


# speedup vs baseline: 14.5953x; 3.9996x over previous
import jax
import jax.numpy as jnp
from jax import lax
from jax.experimental import pallas as pl

_IMAGE_SIZE = (160, 120)
_DEPTH_MIN = 0.4
_DEPTH_MAX = 6.0
_VOXEL_SIZE = 0.03
_GRID_DIMENSIONS = jnp.array([256.0, 256.0, 256.0], dtype=jnp.float32)

_N = 256
_IB = 8


def _gen_frustum(image_size, intrinsic_inv, depth_min, depth_max):
    x = float(image_size[0])
    y = float(image_size[1])
    eight_points = jnp.array([
        [0.0 * depth_min, 0.0 * depth_min, depth_min, 1.0],
        [0.0 * depth_min, y * depth_min, depth_min, 1.0],
        [x * depth_min, y * depth_min, depth_min, 1.0],
        [x * depth_min, 0.0 * depth_min, depth_min, 1.0],
        [0.0 * depth_max, 0.0 * depth_max, depth_max, 1.0],
        [0.0 * depth_max, y * depth_max, depth_max, 1.0],
        [x * depth_max, y * depth_max, depth_max, 1.0],
        [x * depth_max, 0.0 * depth_max, depth_max, 1.0]], dtype=jnp.float32).T
    frustum = (intrinsic_inv @ eight_points).T
    return frustum[:, :3]


def _gen_frustum_volume(frustum, voxel_size):
    maxs = jnp.max(frustum, axis=0) / voxel_size
    mins = jnp.min(frustum, axis=0) / voxel_size
    dims = jnp.ceil(maxs - mins)
    camera2frustum = jnp.array(
        [[1.0 / voxel_size, 0.0, 0.0, 0.0],
         [0.0, 1.0 / voxel_size, 0.0, 0.0],
         [0.0, 0.0, 1.0 / voxel_size, 0.0],
         [0.0, 0.0, 0.0, 1.0]], dtype=jnp.float32)
    camera2frustum = camera2frustum.at[:3, 3].set(-mins)
    return dims, camera2frustum


def _tables(shp, intrinsic):
    intrinsic_inv = jnp.linalg.inv(intrinsic)
    frustum = _gen_frustum(_IMAGE_SIZE, intrinsic_inv, _DEPTH_MIN, _DEPTH_MAX)
    dims, camera2frustum = _gen_frustum_volume(frustum, _VOXEL_SIZE)
    padding = (_GRID_DIMENSIONS - dims) / 2.0
    minv = jnp.linalg.inv(camera2frustum)

    ar = jnp.arange(_N, dtype=jnp.float32)
    g0 = (256.0 - ar) - padding[0] - 1.0
    g1 = (256.0 - ar) - padding[1] - 1.0
    g2 = ar - padding[2] - 1.0
    z = jnp.zeros_like(g0)
    o = jnp.ones_like(g0)
    gh = jnp.concatenate([jnp.stack([g0, z, z, o]),
                          jnp.stack([z, g1, z, o]),
                          jnp.stack([z, z, g2, o])], axis=1)
    pcs = minv @ gh
    pc0, pc1, pc2 = pcs[0, :_N], pcs[1, _N:2 * _N], pcs[2, 2 * _N:]
    cols = jnp.stack([
        jnp.repeat(pc0, _N), jnp.repeat(pc1, _N),
        jnp.tile(pc2, _N), jnp.ones(_N * _N, jnp.float32)])
    dps = intrinsic @ cols
    depth = dps[2].reshape(_N, _N)[:8]
    coordx = (dps[0] / dps[2]).reshape(_N, _N)
    coordy = (dps[1] / dps[2]).reshape(_N, _N)

    dok = (depth[0] >= _DEPTH_MIN) & (depth[0] <= _DEPTH_MAX)
    w = shp.astype(jnp.float32)
    cxok = coordx < w[1]
    cyok = coordy < w[0]
    f32 = jnp.float32
    ckx = (dok[None, :] & cxok).astype(f32)
    cky = cyok.astype(f32)
    kx = (dok[None, :] & cxok & (coordx >= 0.0)).astype(f32)
    ky = (cyok & (coordy >= 0.0)).astype(f32)
    return coordx, coordy, depth, ckx, cky, kx, ky


def _body(mask_ref, cx_ref, cy_ref, d_ref, ckx_ref, cky_ref, kx_ref, ky_ref,
          out_ref, kept_ref):
    mask = mask_ref[0]
    cond = mask & (ckx_ref[0][:, None, :] > 0.5) & (cky_ref[0][None, :, :] > 0.5)
    neg = jnp.float32(-1.0)
    bif = lax.convert_element_type(pl.program_id(0), jnp.float32)
    kf = lax.broadcasted_iota(jnp.int32, (_IB, _N, _N), 2).astype(jnp.float32)
    out_ref[0, :, 0] = jnp.where(cond, bif, neg)
    out_ref[0, :, 1] = jnp.where(cond, cx_ref[0][:, None, :], neg)
    out_ref[0, :, 2] = jnp.where(cond, cy_ref[0][None, :, :], neg)
    out_ref[0, :, 3] = jnp.where(cond, kf, neg)
    out_ref[0, :, 4] = jnp.where(cond, d_ref[0][0][None, None, :], neg)
    kept = mask & (kx_ref[0][:, None, :] > 0.5) & (ky_ref[0][None, :, :] > 0.5)
    kept_ref[0] = kept.astype(jnp.int8)


def kernel(shp, intrinsics, frustum_masks):
    b = intrinsics.shape[0]
    tabs = [_tables(shp, intrinsics[bi]) for bi in range(b)]
    cxt, cyt, dt, ckx, cky, kx, ky = (
        jnp.stack([t[i] for t in tabs]) for i in range(7))

    grid = (b, _N // _IB)
    out, kept = pl.pallas_call(
        _body,
        grid=grid,
        in_specs=[
            pl.BlockSpec((1, _IB, _N, _N), lambda bb, ii: (bb, ii, 0, 0)),
            pl.BlockSpec((1, _IB, _N), lambda bb, ii: (bb, ii, 0)),
            pl.BlockSpec((1, _N, _N), lambda bb, ii: (bb, 0, 0)),
            pl.BlockSpec((1, 8, _N), lambda bb, ii: (bb, 0, 0)),
            pl.BlockSpec((1, _IB, _N), lambda bb, ii: (bb, ii, 0)),
            pl.BlockSpec((1, _N, _N), lambda bb, ii: (bb, 0, 0)),
            pl.BlockSpec((1, _IB, _N), lambda bb, ii: (bb, ii, 0)),
            pl.BlockSpec((1, _N, _N), lambda bb, ii: (bb, 0, 0)),
        ],
        out_specs=[
            pl.BlockSpec((1, _IB, 5, _N, _N), lambda bb, ii: (bb, ii, 0, 0, 0)),
            pl.BlockSpec((1, _IB, _N, _N), lambda bb, ii: (bb, ii, 0, 0)),
        ],
        out_shape=[
            jax.ShapeDtypeStruct((b, _N, 5, _N, _N), jnp.float32),
            jax.ShapeDtypeStruct((b, _N, _N, _N), jnp.int8),
        ],
    )(frustum_masks, cxt, cyt, dt, ckx, cky, kx, ky)
    mappings = jnp.transpose(out, (0, 1, 3, 4, 2))
    return kept.astype(jnp.bool_), mappings
